# Initial kernel scaffold; baseline (speedup 1.0000x reference)
import functools

import jax
import jax.numpy as jnp
from jax import lax
from jax.experimental import pallas as pl
from jax.experimental.pallas import tpu as pltpu
from jax.experimental.pallas import tpu_sc as plsc
from jax._src.pallas import mpmd as _mpmd

T = 4
N = 100000
DIN = 4
HID = 128
DOUT = 96
NCLS = 20
M = 90000
R = 10000
THR = M - R

NW = 32
MP = 90112
CC = MP // NW
PC = 320
MP2 = THR + NW * PC
RP = NW * PC

FB = [0, N, N + RP, N + 2 * RP]
SENT = N + 3 * RP
NV = 131072

SRCL = 100352
GC = SRCL // NW

BM = 1024

_MESH = plsc.VectorSubcoreMesh(core_axis_name="c", subcore_axis_name="s")


def _wid():
    return lax.axis_index("s") * 2 + lax.axis_index("c")


def _iota16():
    return lax.iota(jnp.int32, 16)


def _mlp_body(x_ref, w1_ref, w2_ref, o_ref, *, mask_rows):
    h = jnp.maximum(
        jnp.dot(x_ref[...], w1_ref[...], preferred_element_type=jnp.float32), 0.0
    )
    y = jnp.dot(h, w2_ref[...], preferred_element_type=jnp.float32)
    if mask_rows is not None:
        i = pl.program_id(0)
        rows = i * BM + lax.broadcasted_iota(jnp.int32, y.shape, 0)
        y = jnp.where(rows < mask_rows, y, 0.0)
    o_ref[...] = y


def _mlp(x, w1, w2, out_rows, mask_rows, in_blocks):
    grid = out_rows // BM
    body = functools.partial(_mlp_body, mask_rows=mask_rows)
    return pl.pallas_call(
        body,
        grid=(grid,),
        in_specs=[
            pl.BlockSpec((BM, 8), lambda i: (jnp.minimum(i, in_blocks - 1), 0)),
            pl.BlockSpec((8, HID), lambda i: (0, 0)),
            pl.BlockSpec((HID, DOUT), lambda i: (0, 0)),
        ],
        out_specs=pl.BlockSpec((BM, DOUT), lambda i: (i, 0)),
        out_shape=jax.ShapeDtypeStruct((out_rows, DOUT), jnp.float32),
    )(x, w1, w2)


def _cls_body(v_ref, w_ref, o_ref):
    o_ref[...] = jnp.dot(v_ref[...], w_ref[...], preferred_element_type=jnp.float32)


def _cls(v, wc):
    return pl.pallas_call(
        _cls_body,
        grid=(NV // BM,),
        in_specs=[
            pl.BlockSpec((BM, DOUT), lambda i: (i, 0)),
            pl.BlockSpec((DOUT, NCLS), lambda i: (0, 0)),
        ],
        out_specs=pl.BlockSpec((BM, NCLS), lambda i: (i, 0)),
        out_shape=jax.ShapeDtypeStruct((NV, NCLS), jnp.float32),
    )(v, wc)


def _gf_body(f1, f2, f3, lbi_hbm, gf_out, idx_v, rows_v, sem):
    w = _wid()
    for t, tab in enumerate((f1, f2, f3)):
        base = t * RP + w * PC
        pltpu.sync_copy(lbi_hbm.at[pl.ds(base, PC)], idx_v)
        pltpu.async_copy(tab.at[idx_v], rows_v, sem).wait()
        pltpu.sync_copy(rows_v, gf_out.at[pl.ds(base, PC), :])


def _gather_feats(f1, f2, f3, lbi_flat):
    fn = _mpmd._mpmd_map(
        ((_MESH, _gf_body),),
        (jax.ShapeDtypeStruct((3 * RP, DIN), jnp.float32),),
        scratch_types=(
            pltpu.VMEM((PC,), jnp.int32),
            pltpu.VMEM((PC, DIN), jnp.float32),
            pltpu.SemaphoreType.DMA,
        ),
    )
    return fn(f1, f2, f3, lbi_flat)[0]


def _ct_body(first, fbase, troff, *refs):
    if first:
        (pm_hbm, cm_hbm, tr_hbm, _src_init, _v_in,
         srcnew_hbm, v_hbm,
         pmv, cmv, gv, vv, pv2, g2, pidv, trv, fv, sem) = refs
        srcprev_hbm = None
    else:
        (pm_hbm, cm_hbm, tr_hbm, srcprev_hbm, _src_init, _v_in,
         srcnew_hbm, v_hbm,
         pmv, cmv, gv, vv, pv2, g2, pidv, trv, fv, sem) = refs

    w = _wid()

    base = w * CC
    pltpu.sync_copy(pm_hbm.at[pl.ds(base, CC)], pmv)
    if first:
        g_ref = pmv
    else:
        pltpu.async_copy(srcprev_hbm.at[pmv], gv, sem).wait()
        g_ref = gv
    for k in range(CC // 16):
        i16 = base + k * 16 + _iota16()
        g16 = g_ref[pl.ds(k * 16, 16)]
        vv[pl.ds(k * 16, 16)] = jnp.where(i16 < THR, g16, (fbase - THR) + i16)
    pltpu.sync_copy(cm_hbm.at[pl.ds(base, CC)], cmv)
    pltpu.async_copy(vv, srcnew_hbm.at[cmv], sem).wait()

    base2 = THR + w * PC
    pltpu.sync_copy(pm_hbm.at[pl.ds(base2, PC)], pv2)
    if first:
        g2r = pv2
    else:
        pltpu.async_copy(srcprev_hbm.at[pv2], g2, sem).wait()
        g2r = g2
    for k in range(PC // 16):
        i16 = base2 + k * 16 + _iota16()
        g16 = g2r[pl.ds(k * 16, 16)]
        pidv[pl.ds(k * 16, 16)] = jnp.where(i16 < M, g16, SENT + (i16 & 255))

    pltpu.async_copy(v_hbm.at[pidv], fv, sem).wait()
    pltpu.sync_copy(tr_hbm.at[pl.ds(troff + w * PC, PC), :], trv)

    def fuse_row(j, _):
        for c in range(DOUT // 16):
            sl = (j, pl.ds(c * 16, 16))
            fv[sl] = jnp.maximum(fv[sl] + trv[sl], 0.0)
        return 0

    lax.fori_loop(0, PC, fuse_row, 0)
    pltpu.sync_copy(fv, v_hbm.at[pl.ds(fbase + w * PC, PC), :])


def _compose_fuse(t, pm_row, cm_row, tr, src_prev, src_init, v):
    first = t == 1
    body = functools.partial(_ct_body, first, FB[t], (t - 1) * RP)
    n_in = 5 if first else 6
    fn = _mpmd._mpmd_map(
        ((_MESH, body),),
        (
            jax.ShapeDtypeStruct((SRCL,), jnp.int32),
            jax.ShapeDtypeStruct((NV, DOUT), jnp.float32),
        ),
        input_output_aliases={n_in - 2: 0, n_in - 1: 1},
        scratch_types=(
            pltpu.VMEM((CC,), jnp.int32),
            pltpu.VMEM((CC,), jnp.int32),
            pltpu.VMEM((CC,), jnp.int32),
            pltpu.VMEM((CC,), jnp.int32),
            pltpu.VMEM((PC,), jnp.int32),
            pltpu.VMEM((PC,), jnp.int32),
            pltpu.VMEM((PC,), jnp.int32),
            pltpu.VMEM((PC, DOUT), jnp.float32),
            pltpu.VMEM((PC, DOUT), jnp.float32),
            pltpu.SemaphoreType.DMA,
        ),
    )
    if first:
        return fn(pm_row, cm_row, tr, src_init, v)
    return fn(pm_row, cm_row, tr, src_prev, src_init, v)


def _fin_body(l_hbm, s1, s2, s3, out_hbm, idx_v, rows_v, sem):
    w = _wid()
    base = jnp.minimum(w * GC, N - GC)
    pltpu.sync_copy(l_hbm.at[pl.ds(base, GC), :], rows_v)
    pltpu.sync_copy(rows_v, out_hbm.at[pl.ds(base, GC), :])
    for t, s in ((1, s1), (2, s2), (3, s3)):
        pltpu.sync_copy(s.at[pl.ds(base, GC)], idx_v)
        pltpu.async_copy(l_hbm.at[idx_v], rows_v, sem).wait()
        pltpu.sync_copy(rows_v, out_hbm.at[pl.ds(t * N + base, GC), :])


def _final_gather(logits_tab, s1, s2, s3):
    fn = _mpmd._mpmd_map(
        ((_MESH, _fin_body),),
        (jax.ShapeDtypeStruct((T * N, NCLS), jnp.float32),),
        scratch_types=(
            pltpu.VMEM((GC,), jnp.int32),
            pltpu.VMEM((GC, NCLS), jnp.float32),
            pltpu.SemaphoreType.DMA,
        ),
    )
    return fn(logits_tab, s1, s2, s3)[0]


def kernel(feats, cur_match, prev_match, W_fem1, W_fem2, W_rrm1, W_rrm2, W_cls):
    cm = cur_match.astype(jnp.int32)
    pm = prev_match.astype(jnp.int32)

    pm_p = jnp.pad(pm, ((0, 0), (0, MP2 - M)))
    cm_p = jnp.pad(cm, ((0, 0), (0, MP - M)), constant_values=N)
    lbi_flat = jnp.pad(cm[:, THR:], ((0, 0), (0, RP - R))).reshape(-1)
    feats0p = jnp.pad(feats[0], ((0, 98 * BM - N), (0, 8 - DIN)))
    w_fem1p = jnp.pad(W_fem1, ((0, 8 - DIN), (0, 0)))
    w_rrm1p = jnp.pad(W_rrm1, ((0, 8 - DIN), (0, 0)))
    src_init = SENT + (jnp.arange(SRCL, dtype=jnp.int32) & 255)

    v = _mlp(feats0p, w_fem1p, W_fem2, NV, N, 98)

    gf = _gather_feats(feats[1], feats[2], feats[3], lbi_flat)
    gfp = jnp.pad(gf, ((0, 0), (0, 8 - DIN)))
    tr = _mlp(gfp, w_rrm1p, W_rrm2, 3 * RP, None, 30)

    src1, v = _compose_fuse(1, pm_p[0], cm_p[0], tr, None, src_init, v)
    src2, v = _compose_fuse(2, pm_p[1], cm_p[1], tr, src1, src_init, v)
    src3, v = _compose_fuse(3, pm_p[2], cm_p[2], tr, src2, src_init, v)

    logits_tab = _cls(v, W_cls)

    return _final_gather(logits_tab, src1, src2, src3)


# speedup vs baseline: 1.9986x; 1.9986x over previous
import functools

import jax
import jax.numpy as jnp
from jax import lax
from jax.experimental import pallas as pl
from jax.experimental.pallas import tpu as pltpu
from jax.experimental.pallas import tpu_sc as plsc
from jax._src.pallas import mpmd as _mpmd

T = 4
N = 100000
DIN = 4
HID = 128
DOUT = 96
NCLS = 20
M = 90000
R = 10000
THR = M - R

NW = 32
MP = 90112
CC = MP // NW
PC = 320
MP2 = THR + NW * PC
RP = NW * PC

FB = [0, N, N + RP, N + 2 * RP]
SENT = N + 3 * RP
NV = 131072
LW = 32

SRCL = 100352
GC = SRCL // NW

_MESH = plsc.VectorSubcoreMesh(core_axis_name="c", subcore_axis_name="s")
_SC_PARAMS = pltpu.CompilerParams(needs_layout_passes=False)
_SC_LINEAR = pltpu.CompilerParams(use_tc_tiling_on_sc=False, needs_layout_passes=False)


def _wid():
    return lax.axis_index("s") * 2 + lax.axis_index("c")


def _iota16():
    return lax.iota(jnp.int32, 16)


def _mlp_body(x_ref, w1e_ref, w2_ref, o_ref, *, mask_rows, xb):
    h = jnp.dot(x_ref[...], w1e_ref[...], preferred_element_type=jnp.float32)
    h = jnp.maximum(h, 0.0)
    h = jnp.reshape(h, (32 * xb, HID))
    y = jnp.dot(h, w2_ref[...], preferred_element_type=jnp.float32)
    if mask_rows is not None:
        i = pl.program_id(0)
        rows = i * (32 * xb) + lax.broadcasted_iota(jnp.int32, y.shape, 0)
        y = jnp.where(rows < mask_rows, y, 0.0)
    o_ref[...] = y


def _mlp(x_int, w1e, w2p, out_rows, mask_rows, xb, in_blocks):
    grid = out_rows // (32 * xb)
    body = functools.partial(_mlp_body, mask_rows=mask_rows, xb=xb)
    return pl.pallas_call(
        body,
        grid=(grid,),
        in_specs=[
            pl.BlockSpec((xb, 128), lambda i: (jnp.minimum(i, in_blocks - 1), 0)),
            pl.BlockSpec((128, 32 * HID), lambda i: (0, 0)),
            pl.BlockSpec((HID, 128), lambda i: (0, 0)),
        ],
        out_specs=pl.BlockSpec((32 * xb, 128), lambda i: (i, 0)),
        out_shape=jax.ShapeDtypeStruct((out_rows, 128), jnp.float32),
    )(x_int, w1e, w2p)


def _cls_body(v_ref, w_ref, o_ref):
    v4 = jnp.reshape(v_ref[...], (256, 512))
    o_ref[...] = jnp.dot(v4, w_ref[...], preferred_element_type=jnp.float32)


def _cls(v, wce):
    return pl.pallas_call(
        _cls_body,
        grid=(NV // 1024,),
        in_specs=[
            pl.BlockSpec((1024, 128), lambda i: (i, 0)),
            pl.BlockSpec((512, 128), lambda i: (0, 0)),
        ],
        out_specs=pl.BlockSpec((256, 128), lambda i: (i, 0)),
        out_shape=jax.ShapeDtypeStruct((NV // 4, 128), jnp.float32),
    )(v, wce)


def _gf_body(ff1, ff2, ff3, lbi_hbm, gf_out, lbv, idxb, gout, sem):
    w = _wid()
    for t, ffl in enumerate((ff1, ff2, ff3)):
        base = t * RP + w * PC
        pltpu.sync_copy(lbi_hbm.at[pl.ds(base, PC)], lbv)
        for k in range(PC // 16):
            l16 = lbv[pl.ds(k * 16, 16)]
            pos = k * 64 + _iota16() * 4
            for c in range(DIN):
                plsc.store_scatter(idxb, [pos + c], l16 * 4 + c)
        pltpu.async_copy(ffl.at[idxb], gout, sem).wait()
        pltpu.sync_copy(gout, gf_out.at[pl.ds(base * 4, PC * 4)])


def _gather_feats(ff1, ff2, ff3, lbi_flat):
    fn = _mpmd._mpmd_map(
        ((_MESH, _gf_body),),
        (jax.ShapeDtypeStruct((3 * RP * DIN,), jnp.float32),),
        scratch_types=(
            pltpu.VMEM((PC,), jnp.int32),
            pltpu.VMEM((PC * DIN,), jnp.int32),
            pltpu.VMEM((PC * DIN,), jnp.float32),
            pltpu.SemaphoreType.DMA,
        ),
        compiler_params=_SC_PARAMS,
    )
    return fn(ff1, ff2, ff3, lbi_flat)[0]


def _ct_body(first, fbase, troff, *refs):
    if first:
        (pm_hbm, cm_hbm, tr_hbm, _src_init, _v_in,
         srcnew_hbm, v_hbm,
         pmv, cmv, gv, vv, pv2, g2, pidv, trv, fv, sem) = refs
        srcprev_hbm = None
    else:
        (pm_hbm, cm_hbm, tr_hbm, srcprev_hbm, _src_init, _v_in,
         srcnew_hbm, v_hbm,
         pmv, cmv, gv, vv, pv2, g2, pidv, trv, fv, sem) = refs

    w = _wid()

    base = w * CC
    pltpu.sync_copy(pm_hbm.at[pl.ds(base, CC)], pmv)
    if first:
        g_ref = pmv
    else:
        pltpu.async_copy(srcprev_hbm.at[pmv], gv, sem).wait()
        g_ref = gv
    for k in range(CC // 16):
        i16 = base + k * 16 + _iota16()
        g16 = g_ref[pl.ds(k * 16, 16)]
        vv[pl.ds(k * 16, 16)] = jnp.where(i16 < THR, g16, (fbase - THR) + i16)
    pltpu.sync_copy(cm_hbm.at[pl.ds(base, CC)], cmv)
    pltpu.async_copy(vv, srcnew_hbm.at[cmv], sem).wait()

    base2 = THR + w * PC
    pltpu.sync_copy(pm_hbm.at[pl.ds(base2, PC)], pv2)
    if first:
        g2r = pv2
    else:
        pltpu.async_copy(srcprev_hbm.at[pv2], g2, sem).wait()
        g2r = g2
    for k in range(PC // 16):
        i16 = base2 + k * 16 + _iota16()
        g16 = g2r[pl.ds(k * 16, 16)]
        pidv[pl.ds(k * 16, 16)] = jnp.where(i16 < M, g16, SENT + (i16 & 255))

    pltpu.async_copy(v_hbm.at[pidv], fv, sem).wait()
    pltpu.sync_copy(tr_hbm.at[pl.ds(troff + w * PC, PC), :], trv)

    def fuse_row(j, carry):
        for c in range(128 // 16):
            sl = (j, pl.ds(c * 16, 16))
            fv[sl] = jnp.maximum(fv[sl] + trv[sl], 0.0)
        return carry

    lax.fori_loop(0, PC, fuse_row, 0)
    pltpu.sync_copy(fv, v_hbm.at[pl.ds(fbase + w * PC, PC), :])


def _compose_fuse(t, pm_row, cm_row, tr, src_prev, src_init, v):
    first = t == 1
    body = functools.partial(_ct_body, first, FB[t], (t - 1) * RP)
    n_in = 5 if first else 6
    fn = _mpmd._mpmd_map(
        ((_MESH, body),),
        (
            jax.ShapeDtypeStruct((SRCL,), jnp.int32),
            jax.ShapeDtypeStruct((NV, 128), jnp.float32),
        ),
        input_output_aliases={n_in - 2: 0, n_in - 1: 1},
        scratch_types=(
            pltpu.VMEM((CC,), jnp.int32),
            pltpu.VMEM((CC,), jnp.int32),
            pltpu.VMEM((CC,), jnp.int32),
            pltpu.VMEM((CC,), jnp.int32),
            pltpu.VMEM((PC,), jnp.int32),
            pltpu.VMEM((PC,), jnp.int32),
            pltpu.VMEM((PC,), jnp.int32),
            pltpu.VMEM((PC, 128), jnp.float32),
            pltpu.VMEM((PC, 128), jnp.float32),
            pltpu.SemaphoreType.DMA,
        ),
        compiler_params=_SC_PARAMS,
    )
    if first:
        return fn(pm_row, cm_row, tr, src_init, v)
    return fn(pm_row, cm_row, tr, src_prev, src_init, v)


def _fin_body(l_hbm, s1, s2, s3, out_hbm, idx_v, rows_v, sem):
    w = _wid()
    base = jnp.minimum(w * GC, N - GC)
    pltpu.sync_copy(l_hbm.at[pl.ds(base, GC), :], rows_v)
    pltpu.sync_copy(rows_v.at[:, pl.ds(0, 24)], out_hbm.at[pl.ds(base, GC), :])
    for t, s in ((1, s1), (2, s2), (3, s3)):
        pltpu.sync_copy(s.at[pl.ds(base, GC)], idx_v)
        pltpu.async_copy(l_hbm.at[idx_v], rows_v, sem).wait()
        pltpu.sync_copy(
            rows_v.at[:, pl.ds(0, 24)], out_hbm.at[pl.ds(t * N + base, GC), :]
        )


def _final_gather(logits_tab, s1, s2, s3):
    fn = _mpmd._mpmd_map(
        ((_MESH, _fin_body),),
        (jax.ShapeDtypeStruct((T * N, 24), jnp.float32),),
        scratch_types=(
            pltpu.VMEM((GC,), jnp.int32),
            pltpu.VMEM((GC, LW), jnp.float32),
            pltpu.SemaphoreType.DMA,
        ),
        compiler_params=_SC_LINEAR,
    )
    return fn(logits_tab, s1, s2, s3)[0]


def kernel(feats, cur_match, prev_match, W_fem1, W_fem2, W_rrm1, W_rrm2, W_cls):
    f32 = jnp.float32
    cm = cur_match.astype(jnp.int32)
    pm = prev_match.astype(jnp.int32)

    pm_p = jnp.pad(pm, ((0, 0), (0, MP2 - M)))
    cm_p = jnp.pad(cm, ((0, 0), (0, MP - M)), constant_values=N)
    lbi_flat = jnp.pad(cm[:, THR:], ((0, 0), (0, RP - R))).reshape(-1)
    fflat = jnp.reshape(feats, (T, N * DIN))
    feats0r = jnp.reshape(fflat[0], (N * DIN // 128, 128))
    eye32 = jnp.eye(32, dtype=f32)
    w1e_fem = jnp.kron(eye32, W_fem1)
    w1e_rrm = jnp.kron(eye32, W_rrm1)
    w2p_fem = jnp.pad(W_fem2, ((0, 0), (0, 128 - DOUT)))
    w2p_rrm = jnp.pad(W_rrm2, ((0, 0), (0, 128 - DOUT)))
    wce = jnp.kron(
        jnp.eye(4, dtype=f32),
        jnp.pad(W_cls, ((0, 128 - DOUT), (0, LW - NCLS))),
    )
    src_init = SENT + (jnp.arange(SRCL, dtype=jnp.int32) & 255)

    v = _mlp(feats0r, w1e_fem, w2p_fem, NV, N, 256, 13)

    gf_flat = _gather_feats(fflat[1], fflat[2], fflat[3], lbi_flat)
    gf_r = jnp.reshape(gf_flat, (3 * RP * DIN // 128, 128))
    tr = _mlp(gf_r, w1e_rrm, w2p_rrm, 3 * RP, None, 120, 8)

    src1, v = _compose_fuse(1, pm_p[0], cm_p[0], tr, None, src_init, v)
    src2, v = _compose_fuse(2, pm_p[1], cm_p[1], tr, src1, src_init, v)
    src3, v = _compose_fuse(3, pm_p[2], cm_p[2], tr, src2, src_init, v)

    logits_packed = _cls(v, wce)
    logits_tab = jnp.reshape(logits_packed, (NV, LW))

    out24 = _final_gather(logits_tab, src1, src2, src3)
    return out24[:, :NCLS]


# speedup vs baseline: 2.7139x; 1.3579x over previous
import functools

import jax
import jax.numpy as jnp
from jax import lax
from jax.experimental import pallas as pl
from jax.experimental.pallas import tpu as pltpu
from jax.experimental.pallas import tpu_sc as plsc
from jax._src.pallas import mpmd as _mpmd

T = 4
N = 100000
DIN = 4
HID = 128
DOUT = 96
NCLS = 20
M = 90000
R = 10000
THR = M - R

NW = 32
MP = 90112
CC = MP // NW
PC = 320
MP2 = THR + NW * PC
RP = NW * PC

FB = [0, N, N + RP, N + 2 * RP]
SENT = N + 3 * RP
NV = 131072
LW = 32

SRCL = 100352
GC = SRCL // NW

_MESH = plsc.VectorSubcoreMesh(core_axis_name="c", subcore_axis_name="s")
_SC_PARAMS = pltpu.CompilerParams(needs_layout_passes=False)
_SC_LINEAR = pltpu.CompilerParams(use_tc_tiling_on_sc=False, needs_layout_passes=False)


def _wid():
    return lax.axis_index("s") * 2 + lax.axis_index("c")


def _iota16():
    return lax.iota(jnp.int32, 16)


def _mlp_body(x_ref, w1e_ref, w2_ref, o_ref, *, mask_rows, xb):
    h = jnp.dot(x_ref[...], w1e_ref[...], preferred_element_type=jnp.float32)
    h = jnp.maximum(h, 0.0)
    h = jnp.reshape(h, (32 * xb, HID))
    y = jnp.dot(h, w2_ref[...], preferred_element_type=jnp.float32)
    if mask_rows is not None:
        i = pl.program_id(0)
        rows = i * (32 * xb) + lax.broadcasted_iota(jnp.int32, y.shape, 0)
        y = jnp.where(rows < mask_rows, y, 0.0)
    o_ref[...] = y


def _mlp(x_int, w1e, w2p, out_rows, mask_rows, xb, in_blocks):
    grid = out_rows // (32 * xb)
    body = functools.partial(_mlp_body, mask_rows=mask_rows, xb=xb)
    return pl.pallas_call(
        body,
        grid=(grid,),
        in_specs=[
            pl.BlockSpec((xb, 128), lambda i: (jnp.minimum(i, in_blocks - 1), 0)),
            pl.BlockSpec((128, 32 * HID), lambda i: (0, 0)),
            pl.BlockSpec((HID, 128), lambda i: (0, 0)),
        ],
        out_specs=pl.BlockSpec((32 * xb, 128), lambda i: (i, 0)),
        out_shape=jax.ShapeDtypeStruct((out_rows, 128), jnp.float32),
    )(x_int, w1e, w2p)


def _cls_body(v_ref, w_ref, o_ref):
    v4 = jnp.reshape(v_ref[...], (256, 512))
    o_ref[...] = jnp.dot(v4, w_ref[...], preferred_element_type=jnp.float32)


def _cls(v, wce):
    return pl.pallas_call(
        _cls_body,
        grid=(NV // 1024,),
        in_specs=[
            pl.BlockSpec((1024, 128), lambda i: (i, 0)),
            pl.BlockSpec((512, 128), lambda i: (0, 0)),
        ],
        out_specs=pl.BlockSpec((256, 128), lambda i: (i, 0)),
        out_shape=jax.ShapeDtypeStruct((NV // 4, 128), jnp.float32),
    )(v, wce)


def _gf_body(ff1, ff2, ff3, lbi_hbm, gf_out, lbv, idxb, gout, sem):
    w = _wid()
    for t, ffl in enumerate((ff1, ff2, ff3)):
        base = t * RP + w * PC
        pltpu.sync_copy(lbi_hbm.at[pl.ds(base, PC)], lbv)
        for k in range(PC // 16):
            l16 = lbv[pl.ds(k * 16, 16)]
            pos = k * 64 + _iota16() * 4
            for c in range(DIN):
                plsc.store_scatter(idxb, [pos + c], l16 * 4 + c)
        pltpu.async_copy(ffl.at[idxb], gout, sem).wait()
        pltpu.sync_copy(gout, gf_out.at[pl.ds(base * 4, PC * 4)])


def _gather_feats(ff1, ff2, ff3, lbi_flat):
    fn = _mpmd._mpmd_map(
        ((_MESH, _gf_body),),
        (jax.ShapeDtypeStruct((3 * RP * DIN,), jnp.float32),),
        scratch_types=(
            pltpu.VMEM((PC,), jnp.int32),
            pltpu.VMEM((PC * DIN,), jnp.int32),
            pltpu.VMEM((PC * DIN,), jnp.float32),
            pltpu.SemaphoreType.DMA,
        ),
        compiler_params=_SC_PARAMS,
    )
    return fn(ff1, ff2, ff3, lbi_flat)[0]


def _ct_body(first, fbase, troff, *refs):
    if first:
        (pm_hbm, cm_hbm, tr_hbm, _src_init, _v_in,
         srcnew_hbm, v_hbm,
         pmv, cmv, gv, vv, pv2, g2, pidv, trv, fv,
         sem, sem2, sem3, sem4) = refs
        srcprev_hbm = None
    else:
        (pm_hbm, cm_hbm, tr_hbm, srcprev_hbm, _src_init, _v_in,
         srcnew_hbm, v_hbm,
         pmv, cmv, gv, vv, pv2, g2, pidv, trv, fv,
         sem, sem2, sem3, sem4) = refs

    w = _wid()
    base = w * CC
    base2 = THR + w * PC

    d_pm = pltpu.async_copy(pm_hbm.at[pl.ds(base, CC)], pmv, sem)
    d_pv = pltpu.async_copy(pm_hbm.at[pl.ds(base2, PC)], pv2, sem2)
    d_cm = pltpu.async_copy(cm_hbm.at[pl.ds(base, CC)], cmv, sem3)
    d_tr = pltpu.async_copy(tr_hbm.at[pl.ds(troff + w * PC, PC), :], trv, sem4)

    d_pv.wait()
    if first:
        g2r = pv2
    else:
        pltpu.async_copy(srcprev_hbm.at[pv2], g2, sem2).wait()
        g2r = g2
    for k in range(PC // 16):
        i16 = base2 + k * 16 + _iota16()
        g16 = g2r[pl.ds(k * 16, 16)]
        pidv[pl.ds(k * 16, 16)] = jnp.where(i16 < M, g16, SENT + (i16 & 255))
    d_fv = pltpu.async_copy(v_hbm.at[pidv], fv, sem2)

    d_pm.wait()
    if first:
        g_ref = pmv
    else:
        pltpu.async_copy(srcprev_hbm.at[pmv], gv, sem).wait()
        g_ref = gv
    for k in range(CC // 16):
        i16 = base + k * 16 + _iota16()
        g16 = g_ref[pl.ds(k * 16, 16)]
        vv[pl.ds(k * 16, 16)] = jnp.where(i16 < THR, g16, (fbase - THR) + i16)
    d_cm.wait()
    d_sc = pltpu.async_copy(vv, srcnew_hbm.at[cmv], sem3)

    d_fv.wait()
    d_tr.wait()

    def fuse_row(j, carry):
        for c in range(128 // 16):
            sl = (j, pl.ds(c * 16, 16))
            fv[sl] = jnp.maximum(fv[sl] + trv[sl], 0.0)
        return carry

    lax.fori_loop(0, PC, fuse_row, 0)
    pltpu.async_copy(fv, v_hbm.at[pl.ds(fbase + w * PC, PC), :], sem4).wait()
    d_sc.wait()


def _compose_fuse(t, pm_row, cm_row, tr, src_prev, src_init, v):
    first = t == 1
    body = functools.partial(_ct_body, first, FB[t], (t - 1) * RP)
    n_in = 5 if first else 6
    fn = _mpmd._mpmd_map(
        ((_MESH, body),),
        (
            jax.ShapeDtypeStruct((SRCL,), jnp.int32),
            jax.ShapeDtypeStruct((NV, 128), jnp.float32),
        ),
        input_output_aliases={n_in - 2: 0, n_in - 1: 1},
        scratch_types=(
            pltpu.VMEM((CC,), jnp.int32),
            pltpu.VMEM((CC,), jnp.int32),
            pltpu.VMEM((CC,), jnp.int32),
            pltpu.VMEM((CC,), jnp.int32),
            pltpu.VMEM((PC,), jnp.int32),
            pltpu.VMEM((PC,), jnp.int32),
            pltpu.VMEM((PC,), jnp.int32),
            pltpu.VMEM((PC, 128), jnp.float32),
            pltpu.VMEM((PC, 128), jnp.float32),
            pltpu.SemaphoreType.DMA,
            pltpu.SemaphoreType.DMA,
            pltpu.SemaphoreType.DMA,
            pltpu.SemaphoreType.DMA,
        ),
        compiler_params=_SC_PARAMS,
    )
    if first:
        return fn(pm_row, cm_row, tr, src_init, v)
    return fn(pm_row, cm_row, tr, src_prev, src_init, v)


GH = GC // 2


def _repack(rows_v, rows20):
    def grp(g, carry):
        for q in range(5):
            o16 = g * 80 + q * 16 + _iota16()
            j16 = o16 // 20
            c16 = o16 - j16 * 20
            v = plsc.load_gather(rows_v, [j16, c16])
            plsc.store_scatter(rows20, [j16, c16], v)
        return carry

    lax.fori_loop(0, GH * 20 // 80, grp, 0)


def _fin_body(l_hbm, s1, s2, s3, out_hbm, idx_v, rows_v, rows20, sem):
    w = _wid()
    base0 = jnp.minimum(w * GC, N - GC)
    for half in range(2):
        base = base0 + half * GH
        pltpu.sync_copy(l_hbm.at[pl.ds(base, GH), :], rows_v)
        _repack(rows_v, rows20)
        pltpu.sync_copy(rows20, out_hbm.at[pl.ds(base, GH), :])
        for t, s in ((1, s1), (2, s2), (3, s3)):
            pltpu.sync_copy(s.at[pl.ds(base, GH)], idx_v)
            pltpu.async_copy(l_hbm.at[idx_v], rows_v, sem).wait()
            _repack(rows_v, rows20)
            pltpu.sync_copy(rows20, out_hbm.at[pl.ds(t * N + base, GH), :])


def _final_gather(logits_tab, s1, s2, s3):
    fn = _mpmd._mpmd_map(
        ((_MESH, _fin_body),),
        (jax.ShapeDtypeStruct((T * N, NCLS), jnp.float32),),
        scratch_types=(
            pltpu.VMEM((GH,), jnp.int32),
            pltpu.VMEM((GH, LW), jnp.float32),
            pltpu.VMEM((GH, NCLS), jnp.float32),
            pltpu.SemaphoreType.DMA,
        ),
        compiler_params=_SC_LINEAR,
    )
    return fn(logits_tab, s1, s2, s3)[0]


def kernel(feats, cur_match, prev_match, W_fem1, W_fem2, W_rrm1, W_rrm2, W_cls):
    f32 = jnp.float32
    cm = cur_match.astype(jnp.int32)
    pm = prev_match.astype(jnp.int32)

    pm_p = jnp.pad(pm, ((0, 0), (0, MP2 - M)))
    cm_p = jnp.pad(cm, ((0, 0), (0, MP - M)), constant_values=N)
    lbi_flat = jnp.pad(cm[:, THR:], ((0, 0), (0, RP - R))).reshape(-1)
    feats0r = jnp.reshape(feats[0], (N * DIN // 128, 128))
    ffl1 = jnp.reshape(feats[1], (N * DIN,))
    ffl2 = jnp.reshape(feats[2], (N * DIN,))
    ffl3 = jnp.reshape(feats[3], (N * DIN,))
    eye32 = jnp.eye(32, dtype=f32)
    w1e_fem = jnp.kron(eye32, W_fem1)
    w1e_rrm = jnp.kron(eye32, W_rrm1)
    w2p_fem = jnp.pad(W_fem2, ((0, 0), (0, 128 - DOUT)))
    w2p_rrm = jnp.pad(W_rrm2, ((0, 0), (0, 128 - DOUT)))
    wce = jnp.kron(
        jnp.eye(4, dtype=f32),
        jnp.pad(W_cls, ((0, 128 - DOUT), (0, LW - NCLS))),
    )
    src_init = SENT + (jnp.arange(SRCL, dtype=jnp.int32) & 255)

    v = _mlp(feats0r, w1e_fem, w2p_fem, NV, N, 256, 13)

    gf_flat = _gather_feats(ffl1, ffl2, ffl3, lbi_flat)
    gf_r = jnp.reshape(gf_flat, (3 * RP * DIN // 128, 128))
    tr = _mlp(gf_r, w1e_rrm, w2p_rrm, 3 * RP, None, 120, 8)

    src1, v = _compose_fuse(1, pm_p[0], cm_p[0], tr, None, src_init, v)
    src2, v = _compose_fuse(2, pm_p[1], cm_p[1], tr, src1, src_init, v)
    src3, v = _compose_fuse(3, pm_p[2], cm_p[2], tr, src2, src_init, v)

    logits_packed = _cls(v, wce)
    logits_tab = jnp.reshape(logits_packed, (NV, LW))

    return _final_gather(logits_tab, src1, src2, src3)


# speedup vs baseline: 3.1030x; 1.1434x over previous
import functools

import jax
import jax.numpy as jnp
from jax import lax
from jax.experimental import pallas as pl
from jax.experimental.pallas import tpu as pltpu
from jax.experimental.pallas import tpu_sc as plsc
from jax._src.pallas import mpmd as _mpmd

T = 4
N = 100000
DIN = 4
HID = 128
DOUT = 96
NCLS = 20
M = 90000
R = 10000
THR = M - R

NW = 32
MP = 90112
CC = MP // NW
PC = 320
MP2 = THR + NW * PC
RP = NW * PC

FB = [0, N, N + RP, N + 2 * RP]
SENT = N + 3 * RP
NV = 131072
LW = 32

SRCL = 100352
GC = SRCL // NW

_MESH = plsc.VectorSubcoreMesh(core_axis_name="c", subcore_axis_name="s")
_SC_PARAMS = pltpu.CompilerParams(needs_layout_passes=False)
_SC_LINEAR = pltpu.CompilerParams(use_tc_tiling_on_sc=False, needs_layout_passes=False)


def _wid():
    return lax.axis_index("s") * 2 + lax.axis_index("c")


def _iota16():
    return lax.iota(jnp.int32, 16)


def _mlp_body(x_ref, w1e_ref, w2_ref, o_ref, *, mask_rows, xb):
    h = jnp.dot(x_ref[...], w1e_ref[...], preferred_element_type=jnp.float32)
    h = jnp.maximum(h, 0.0)
    h = jnp.reshape(h, (32 * xb, HID))
    y = jnp.dot(h, w2_ref[...], preferred_element_type=jnp.float32)
    if mask_rows is not None:
        i = pl.program_id(0)
        rows = i * (32 * xb) + lax.broadcasted_iota(jnp.int32, y.shape, 0)
        y = jnp.where(rows < mask_rows, y, 0.0)
    o_ref[...] = y


def _fem_body(x_ref, w1_ref, w2_ref, wce_ref, o_ref, o2_ref):
    h = jnp.maximum(
        jnp.dot(x_ref[...], w1_ref[...], preferred_element_type=jnp.float32), 0.0
    )
    y = jnp.dot(h, w2_ref[...], preferred_element_type=jnp.float32)
    i = pl.program_id(0)
    rows = i * BMF + lax.broadcasted_iota(jnp.int32, y.shape, 0)
    y = jnp.where(rows < N, y, 0.0)
    o_ref[...] = y
    v4 = jnp.reshape(y, (BMF // 4, 512))
    o2_ref[...] = jnp.dot(v4, wce_ref[...], preferred_element_type=jnp.float32)


BMF = 2048


def _fem(x, w1p, w2p, wce):
    nb = (N + BMF - 1) // BMF
    return pl.pallas_call(
        _fem_body,
        grid=(NV // BMF,),
        in_specs=[
            pl.BlockSpec((BMF, 8), lambda i: (jnp.minimum(i, nb - 1), 0)),
            pl.BlockSpec((8, HID), lambda i: (0, 0)),
            pl.BlockSpec((HID, 128), lambda i: (0, 0)),
            pl.BlockSpec((512, 128), lambda i: (0, 0)),
        ],
        out_specs=[
            pl.BlockSpec((BMF, 128), lambda i: (i, 0)),
            pl.BlockSpec((BMF // 4, 128), lambda i: (i, 0)),
        ],
        out_shape=[
            jax.ShapeDtypeStruct((NV, 128), jnp.float32),
            jax.ShapeDtypeStruct((NV // 4, 128), jnp.float32),
        ],
    )(x, w1p, w2p, wce)


def _mlp(x_int, w1e, w2p, out_rows, mask_rows, xb, in_blocks):
    grid = out_rows // (32 * xb)
    body = functools.partial(_mlp_body, mask_rows=mask_rows, xb=xb)
    return pl.pallas_call(
        body,
        grid=(grid,),
        in_specs=[
            pl.BlockSpec((xb, 128), lambda i: (jnp.minimum(i, in_blocks - 1), 0)),
            pl.BlockSpec((128, 32 * HID), lambda i: (0, 0)),
            pl.BlockSpec((HID, 128), lambda i: (0, 0)),
        ],
        out_specs=pl.BlockSpec((32 * xb, 128), lambda i: (i, 0)),
        out_shape=jax.ShapeDtypeStruct((out_rows, 128), jnp.float32),
    )(x_int, w1e, w2p)


def _cls_body(v_ref, w_ref, lp_ref, o_ref):
    del lp_ref
    v4 = jnp.reshape(v_ref[...], (256, 512))
    o_ref[...] = jnp.dot(v4, w_ref[...], preferred_element_type=jnp.float32)


CLS0 = 96


def _cls(v, wce, lp0):
    return pl.pallas_call(
        _cls_body,
        grid=(NV // 1024 - CLS0,),
        in_specs=[
            pl.BlockSpec((1024, 128), lambda i: (CLS0 + i, 0)),
            pl.BlockSpec((512, 128), lambda i: (0, 0)),
            pl.BlockSpec((256, 128), lambda i: (CLS0 + i, 0)),
        ],
        out_specs=pl.BlockSpec((256, 128), lambda i: (CLS0 + i, 0)),
        out_shape=jax.ShapeDtypeStruct((NV // 4, 128), jnp.float32),
        input_output_aliases={2: 0},
    )(v, wce, lp0)


def _gf_body(ff1, ff2, ff3, lbi_hbm, gf_out, lbv, idxb, gout, sem):
    w = _wid()
    for t, ffl in enumerate((ff1, ff2, ff3)):
        base = t * RP + w * PC
        pltpu.sync_copy(lbi_hbm.at[pl.ds(base, PC)], lbv)
        for k in range(PC // 16):
            l16 = lbv[pl.ds(k * 16, 16)]
            pos = k * 64 + _iota16() * 4
            for c in range(DIN):
                plsc.store_scatter(idxb, [pos + c], l16 * 4 + c)
        pltpu.async_copy(ffl.at[idxb], gout, sem).wait()
        pltpu.sync_copy(gout, gf_out.at[pl.ds(base * 4, PC * 4)])


def _gather_feats(ff1, ff2, ff3, lbi_flat):
    fn = _mpmd._mpmd_map(
        ((_MESH, _gf_body),),
        (jax.ShapeDtypeStruct((3 * RP * DIN,), jnp.float32),),
        scratch_types=(
            pltpu.VMEM((PC,), jnp.int32),
            pltpu.VMEM((PC * DIN,), jnp.int32),
            pltpu.VMEM((PC * DIN,), jnp.float32),
            pltpu.SemaphoreType.DMA,
        ),
        compiler_params=_SC_PARAMS,
    )
    return fn(ff1, ff2, ff3, lbi_flat)[0]


def _ct_body(first, fbase, troff, *refs):
    if first:
        (pm_hbm, cm_hbm, tr_hbm, _src_init, _v_in,
         srcnew_hbm, v_hbm,
         pmv, cmv, gv, vv, pv2, g2, pidv, trv, fv,
         sem, sem2, sem3, sem4) = refs
        srcprev_hbm = None
    else:
        (pm_hbm, cm_hbm, tr_hbm, srcprev_hbm, _src_init, _v_in,
         srcnew_hbm, v_hbm,
         pmv, cmv, gv, vv, pv2, g2, pidv, trv, fv,
         sem, sem2, sem3, sem4) = refs

    w = _wid()
    base = w * CC
    base2 = THR + w * PC

    d_pm = pltpu.async_copy(pm_hbm.at[pl.ds(base, CC)], pmv, sem)
    d_pv = pltpu.async_copy(pm_hbm.at[pl.ds(base2, PC)], pv2, sem2)
    d_cm = pltpu.async_copy(cm_hbm.at[pl.ds(base, CC)], cmv, sem3)
    d_tr = pltpu.async_copy(tr_hbm.at[pl.ds(troff + w * PC, PC), :], trv, sem4)

    d_pv.wait()
    if first:
        g2r = pv2
    else:
        pltpu.async_copy(srcprev_hbm.at[pv2], g2, sem2).wait()
        g2r = g2
    for k in range(PC // 16):
        i16 = base2 + k * 16 + _iota16()
        g16 = g2r[pl.ds(k * 16, 16)]
        pidv[pl.ds(k * 16, 16)] = jnp.where(i16 < M, g16, SENT + (i16 & 255))
    d_fv = pltpu.async_copy(v_hbm.at[pidv], fv, sem2)

    d_pm.wait()
    if first:
        g_ref = pmv
    else:
        pltpu.async_copy(srcprev_hbm.at[pmv], gv, sem).wait()
        g_ref = gv
    for k in range(CC // 16):
        i16 = base + k * 16 + _iota16()
        g16 = g_ref[pl.ds(k * 16, 16)]
        vv[pl.ds(k * 16, 16)] = jnp.where(i16 < THR, g16, (fbase - THR) + i16)
    d_cm.wait()
    d_sc = pltpu.async_copy(vv, srcnew_hbm.at[cmv], sem3)

    d_fv.wait()
    d_tr.wait()

    def fuse_row(j, carry):
        for c in range(128 // 16):
            sl = (j, pl.ds(c * 16, 16))
            fv[sl] = jnp.maximum(fv[sl] + trv[sl], 0.0)
        return carry

    lax.fori_loop(0, PC, fuse_row, 0)
    pltpu.async_copy(fv, v_hbm.at[pl.ds(fbase + w * PC, PC), :], sem4).wait()
    d_sc.wait()


def _compose_fuse(t, pm_row, cm_row, tr, src_prev, src_init, v):
    first = t == 1
    body = functools.partial(_ct_body, first, FB[t], (t - 1) * RP)
    n_in = 5 if first else 6
    fn = _mpmd._mpmd_map(
        ((_MESH, body),),
        (
            jax.ShapeDtypeStruct((SRCL,), jnp.int32),
            jax.ShapeDtypeStruct((NV, 128), jnp.float32),
        ),
        input_output_aliases={n_in - 2: 0, n_in - 1: 1},
        scratch_types=(
            pltpu.VMEM((CC,), jnp.int32),
            pltpu.VMEM((CC,), jnp.int32),
            pltpu.VMEM((CC,), jnp.int32),
            pltpu.VMEM((CC,), jnp.int32),
            pltpu.VMEM((PC,), jnp.int32),
            pltpu.VMEM((PC,), jnp.int32),
            pltpu.VMEM((PC,), jnp.int32),
            pltpu.VMEM((PC, 128), jnp.float32),
            pltpu.VMEM((PC, 128), jnp.float32),
            pltpu.SemaphoreType.DMA,
            pltpu.SemaphoreType.DMA,
            pltpu.SemaphoreType.DMA,
            pltpu.SemaphoreType.DMA,
        ),
        compiler_params=_SC_PARAMS,
    )
    if first:
        return fn(pm_row, cm_row, tr, src_init, v)
    return fn(pm_row, cm_row, tr, src_prev, src_init, v)


SRCP = 112896
TPW = 98
PW = TPW * 128
TSUB = 7
SUBP = TSUB * 128
NSUB = TPW // TSUB
NTILE = T * N // 128


def _fin_body(l_hbm, sall_hbm, o_hbm, bufa, bufb, idsv, rows, tbuf, sem, sem2):
    w = _wid()
    tile0 = jnp.minimum(w * TPW, NTILE - TPW)
    p_lo = tile0 * 128
    t_lo = ((p_lo >= N).astype(jnp.int32) + (p_lo >= 2 * N).astype(jnp.int32)
            + (p_lo >= 3 * N).astype(jnp.int32))
    i_lo = p_lo - t_lo * N
    tbound = (t_lo + 1) * N
    shift = tbound - p_lo
    tb = jnp.minimum(t_lo + 1, 3)
    da = pltpu.async_copy(sall_hbm.at[pl.ds(t_lo * SRCP + i_lo, PW)], bufa, sem)
    db = pltpu.async_copy(sall_hbm.at[pl.ds(tb * SRCP, PW)], bufb, sem2)
    da.wait()
    db.wait()
    for g in range(PW // 16):
        k16 = g * 16 + _iota16()
        p16 = p_lo + k16
        a16 = bufa[pl.ds(g * 16, 16)]
        b16 = plsc.load_gather(bufb, [jnp.maximum(k16 - shift, 0)])
        idsv[pl.ds(g * 16, 16)] = jnp.where(p16 < tbound, a16, b16)
    for sub in range(NSUB):
        pltpu.async_copy(
            l_hbm.at[idsv.at[pl.ds(sub * SUBP, SUBP)]], rows, sem
        ).wait()

        def col_blk(blk, carry):
            def col_c(c, carry2):
                cr = c // 8
                cc = c - cr * 8
                for lv in range(8):
                    lane16 = lv * 16 + _iota16()
                    v = plsc.load_gather(rows, [blk * 128 + lane16, c + 0 * lane16])
                    tbuf[cr, pl.ds(blk * 1024 + cc * 128 + lv * 16, 16)] = v
                return carry2

            return lax.fori_loop(0, 24, col_c, carry)

        lax.fori_loop(0, TSUB, col_blk, 0)
        pltpu.sync_copy(
            tbuf, o_hbm.at[:, pl.ds((tile0 + sub * TSUB) * 1024, TSUB * 1024)]
        )


def _final_gather(logits_tab, src_all):
    fn = _mpmd._mpmd_map(
        ((_MESH, _fin_body),),
        (jax.ShapeDtypeStruct((3, NTILE * 1024), jnp.float32),),
        scratch_types=(
            pltpu.VMEM((PW,), jnp.int32),
            pltpu.VMEM((PW,), jnp.int32),
            pltpu.VMEM((PW,), jnp.int32),
            pltpu.VMEM((SUBP, LW), jnp.float32),
            pltpu.VMEM((3, TSUB * 1024), jnp.float32),
            pltpu.SemaphoreType.DMA,
            pltpu.SemaphoreType.DMA,
        ),
        compiler_params=_SC_LINEAR,
    )
    return fn(logits_tab, src_all)[0]


def kernel(feats, cur_match, prev_match, W_fem1, W_fem2, W_rrm1, W_rrm2, W_cls):
    f32 = jnp.float32
    cm = cur_match.astype(jnp.int32)
    pm = prev_match.astype(jnp.int32)

    pm_p = jnp.pad(pm, ((0, 0), (0, MP2 - M)))
    cm_p = jnp.pad(cm, ((0, 0), (0, MP - M)), constant_values=N)
    lbi_flat = jnp.pad(cm[:, THR:], ((0, 0), (0, RP - R))).reshape(-1)
    feats0p = jnp.pad(feats[0], ((0, BMF * (NV // BMF) - N), (0, 8 - DIN)))
    w1p_fem = jnp.pad(W_fem1, ((0, 8 - DIN), (0, 0)))
    w1e_rrm = jnp.kron(jnp.eye(32, dtype=f32), W_rrm1)
    w2p_fem = jnp.pad(W_fem2, ((0, 0), (0, 128 - DOUT)))
    w2p_rrm = jnp.pad(W_rrm2, ((0, 0), (0, 128 - DOUT)))
    wce = jnp.kron(
        jnp.eye(4, dtype=f32),
        jnp.pad(W_cls, ((0, 128 - DOUT), (0, LW - NCLS))),
    )
    src_init = SENT + (jnp.arange(SRCL, dtype=jnp.int32) & 255)

    v, lp0 = _fem(feats0p, w1p_fem, w2p_fem, wce)

    ffl1 = jnp.reshape(feats[1], (N * DIN,))
    ffl2 = jnp.reshape(feats[2], (N * DIN,))
    ffl3 = jnp.reshape(feats[3], (N * DIN,))
    gf_flat = _gather_feats(ffl1, ffl2, ffl3, lbi_flat)
    gf_r = jnp.reshape(gf_flat, (3 * RP * DIN // 128, 128))
    tr = _mlp(gf_r, w1e_rrm, w2p_rrm, 3 * RP, None, 120, 8)

    src1, v = _compose_fuse(1, pm_p[0], cm_p[0], tr, None, src_init, v)
    src2, v = _compose_fuse(2, pm_p[1], cm_p[1], tr, src1, src_init, v)
    src3, v = _compose_fuse(3, pm_p[2], cm_p[2], tr, src2, src_init, v)

    logits_packed = _cls(v, wce, lp0)
    logits_tab = jnp.reshape(logits_packed, (NV, LW))

    src_all = jnp.concatenate([
        jnp.arange(SRCP, dtype=jnp.int32),
        jnp.pad(src1, (0, SRCP - SRCL)),
        jnp.pad(src2, (0, SRCP - SRCL)),
        jnp.pad(src3, (0, SRCP - SRCL)),
    ])
    obuf = _final_gather(logits_tab, src_all).reshape(3, NTILE, 8, 128)
    out = jnp.transpose(obuf, (1, 3, 0, 2)).reshape(T * N, 24)
    return out[:, :NCLS]


# speedup vs baseline: 3.2155x; 1.0362x over previous
import functools

import jax
import jax.numpy as jnp
from jax import lax
from jax.experimental import pallas as pl
from jax.experimental.pallas import tpu as pltpu
from jax.experimental.pallas import tpu_sc as plsc
from jax._src.pallas import mpmd as _mpmd

T = 4
N = 100000
DIN = 4
HID = 128
DOUT = 96
NCLS = 20
M = 90000
R = 10000
THR = M - R

NW = 32
MP = 90112
CC = MP // NW
PC = 320
MP2 = THR + NW * PC
RP = NW * PC

FB = [0, N, N + RP, N + 2 * RP]
SENT = N + 3 * RP
NV = 131072
LW = 32

SRCL = 100352
GC = SRCL // NW

_MESH = plsc.VectorSubcoreMesh(core_axis_name="c", subcore_axis_name="s")
_SC_PARAMS = pltpu.CompilerParams(needs_layout_passes=False)
_SC_LINEAR = pltpu.CompilerParams(use_tc_tiling_on_sc=False, needs_layout_passes=False)


def _wid():
    return lax.axis_index("s") * 2 + lax.axis_index("c")


def _iota16():
    return lax.iota(jnp.int32, 16)


def _mlp_body(x_ref, w1e_ref, w2_ref, o_ref, *, mask_rows, xb):
    h = jnp.dot(x_ref[...], w1e_ref[...], preferred_element_type=jnp.float32)
    h = jnp.maximum(h, 0.0)
    h = jnp.reshape(h, (32 * xb, HID))
    y = jnp.dot(h, w2_ref[...], preferred_element_type=jnp.float32)
    if mask_rows is not None:
        i = pl.program_id(0)
        rows = i * (32 * xb) + lax.broadcasted_iota(jnp.int32, y.shape, 0)
        y = jnp.where(rows < mask_rows, y, 0.0)
    o_ref[...] = y


def _fem_body(x_ref, w1_ref, w2_ref, wce_ref, o_ref, o2_ref):
    h = jnp.maximum(
        jnp.dot(x_ref[...], w1_ref[...], preferred_element_type=jnp.float32), 0.0
    )
    y = jnp.dot(h, w2_ref[...], preferred_element_type=jnp.float32)
    i = pl.program_id(0)
    rows = i * BMF + lax.broadcasted_iota(jnp.int32, y.shape, 0)
    y = jnp.where(rows < N, y, 0.0)
    o_ref[...] = y
    v4 = jnp.reshape(y, (BMF // 4, 512))
    o2_ref[...] = jnp.dot(v4, wce_ref[...], preferred_element_type=jnp.float32)


BMF = 2048


def _fem(x, w1p, w2p, wce):
    nb = (N + BMF - 1) // BMF
    return pl.pallas_call(
        _fem_body,
        grid=(NV // BMF,),
        in_specs=[
            pl.BlockSpec((BMF, 8), lambda i: (jnp.minimum(i, nb - 1), 0)),
            pl.BlockSpec((8, HID), lambda i: (0, 0)),
            pl.BlockSpec((HID, 128), lambda i: (0, 0)),
            pl.BlockSpec((512, 128), lambda i: (0, 0)),
        ],
        out_specs=[
            pl.BlockSpec((BMF, 128), lambda i: (i, 0)),
            pl.BlockSpec((BMF // 4, 128), lambda i: (i, 0)),
        ],
        out_shape=[
            jax.ShapeDtypeStruct((NV, 128), jnp.float32),
            jax.ShapeDtypeStruct((NV // 4, 128), jnp.float32),
        ],
    )(x, w1p, w2p, wce)


def _mlp(x_int, w1e, w2p, out_rows, mask_rows, xb, in_blocks):
    grid = out_rows // (32 * xb)
    body = functools.partial(_mlp_body, mask_rows=mask_rows, xb=xb)
    return pl.pallas_call(
        body,
        grid=(grid,),
        in_specs=[
            pl.BlockSpec((xb, 128), lambda i: (jnp.minimum(i, in_blocks - 1), 0)),
            pl.BlockSpec((128, 32 * HID), lambda i: (0, 0)),
            pl.BlockSpec((HID, 128), lambda i: (0, 0)),
        ],
        out_specs=pl.BlockSpec((32 * xb, 128), lambda i: (i, 0)),
        out_shape=jax.ShapeDtypeStruct((out_rows, 128), jnp.float32),
    )(x_int, w1e, w2p)


def _cls_body(v_ref, w_ref, lp_ref, o_ref):
    del lp_ref
    v4 = jnp.reshape(v_ref[...], (256, 512))
    o_ref[...] = jnp.dot(v4, w_ref[...], preferred_element_type=jnp.float32)


CLS0 = 96


def _cls(v, wce, lp0):
    return pl.pallas_call(
        _cls_body,
        grid=(NV // 1024 - CLS0,),
        in_specs=[
            pl.BlockSpec((1024, 128), lambda i: (CLS0 + i, 0)),
            pl.BlockSpec((512, 128), lambda i: (0, 0)),
            pl.BlockSpec((256, 128), lambda i: (CLS0 + i, 0)),
        ],
        out_specs=pl.BlockSpec((256, 128), lambda i: (CLS0 + i, 0)),
        out_shape=jax.ShapeDtypeStruct((NV // 4, 128), jnp.float32),
        input_output_aliases={2: 0},
    )(v, wce, lp0)


def _gf_body(ff1, ff2, ff3, lbi_hbm, gf_out, lbv, idxb, gout, sem):
    w = _wid()
    for t, ffl in enumerate((ff1, ff2, ff3)):
        base = t * RP + w * PC
        pltpu.sync_copy(lbi_hbm.at[pl.ds(base, PC)], lbv)
        for k in range(PC // 16):
            l16 = lbv[pl.ds(k * 16, 16)]
            pos = k * 64 + _iota16() * 4
            for c in range(DIN):
                plsc.store_scatter(idxb, [pos + c], l16 * 4 + c)
        pltpu.async_copy(ffl.at[idxb], gout, sem).wait()
        pltpu.sync_copy(gout, gf_out.at[pl.ds(base * 4, PC * 4)])


def _gather_feats(ff1, ff2, ff3, lbi_flat):
    fn = _mpmd._mpmd_map(
        ((_MESH, _gf_body),),
        (jax.ShapeDtypeStruct((3 * RP * DIN,), jnp.float32),),
        scratch_types=(
            pltpu.VMEM((PC,), jnp.int32),
            pltpu.VMEM((PC * DIN,), jnp.int32),
            pltpu.VMEM((PC * DIN,), jnp.float32),
            pltpu.SemaphoreType.DMA,
        ),
        compiler_params=_SC_PARAMS,
    )
    return fn(ff1, ff2, ff3, lbi_flat)[0]


def _ct_body(first, fbase, troff, *refs):
    if first:
        (pm_hbm, cm_hbm, tr_hbm, _src_init, _v_in,
         srcnew_hbm, v_hbm,
         pmv, cmv, gv, vv, pv2, g2, pidv, trv, fv,
         sem, sem2, sem3, sem4) = refs
        srcprev_hbm = None
    else:
        (pm_hbm, cm_hbm, tr_hbm, srcprev_hbm, _src_init, _v_in,
         srcnew_hbm, v_hbm,
         pmv, cmv, gv, vv, pv2, g2, pidv, trv, fv,
         sem, sem2, sem3, sem4) = refs

    w = _wid()
    base = w * CC
    base2 = THR + w * PC

    d_pm = pltpu.async_copy(pm_hbm.at[pl.ds(base, CC)], pmv, sem)
    d_pv = pltpu.async_copy(pm_hbm.at[pl.ds(base2, PC)], pv2, sem2)
    d_cm = pltpu.async_copy(cm_hbm.at[pl.ds(base, CC)], cmv, sem3)
    d_tr = pltpu.async_copy(tr_hbm.at[pl.ds(troff + w * PC, PC), :], trv, sem4)

    d_pv.wait()
    if first:
        g2r = pv2
    else:
        pltpu.async_copy(srcprev_hbm.at[pv2], g2, sem2).wait()
        g2r = g2
    for k in range(PC // 16):
        i16 = base2 + k * 16 + _iota16()
        g16 = g2r[pl.ds(k * 16, 16)]
        pidv[pl.ds(k * 16, 16)] = jnp.where(i16 < M, g16, SENT + (i16 & 255))
    d_fv = pltpu.async_copy(v_hbm.at[pidv], fv, sem2)

    d_pm.wait()
    if first:
        g_ref = pmv
    else:
        pltpu.async_copy(srcprev_hbm.at[pmv], gv, sem).wait()
        g_ref = gv
    for k in range(CC // 16):
        i16 = base + k * 16 + _iota16()
        g16 = g_ref[pl.ds(k * 16, 16)]
        vv[pl.ds(k * 16, 16)] = jnp.where(i16 < THR, g16, (fbase - THR) + i16)
    d_cm.wait()
    d_sc = pltpu.async_copy(vv, srcnew_hbm.at[cmv], sem3)

    d_fv.wait()
    d_tr.wait()

    def fuse_row(j, carry):
        for c in range(128 // 16):
            sl = (j, pl.ds(c * 16, 16))
            fv[sl] = jnp.maximum(fv[sl] + trv[sl], 0.0)
        return carry

    lax.fori_loop(0, PC, fuse_row, 0)
    pltpu.async_copy(fv, v_hbm.at[pl.ds(fbase + w * PC, PC), :], sem4).wait()
    d_sc.wait()


def _compose_fuse(t, pm_row, cm_row, tr, src_prev, src_init, v):
    first = t == 1
    body = functools.partial(_ct_body, first, FB[t], (t - 1) * RP)
    n_in = 5 if first else 6
    fn = _mpmd._mpmd_map(
        ((_MESH, body),),
        (
            jax.ShapeDtypeStruct((SRCL,), jnp.int32),
            jax.ShapeDtypeStruct((NV, 128), jnp.float32),
        ),
        input_output_aliases={n_in - 2: 0, n_in - 1: 1},
        scratch_types=(
            pltpu.VMEM((CC,), jnp.int32),
            pltpu.VMEM((CC,), jnp.int32),
            pltpu.VMEM((CC,), jnp.int32),
            pltpu.VMEM((CC,), jnp.int32),
            pltpu.VMEM((PC,), jnp.int32),
            pltpu.VMEM((PC,), jnp.int32),
            pltpu.VMEM((PC,), jnp.int32),
            pltpu.VMEM((PC, 128), jnp.float32),
            pltpu.VMEM((PC, 128), jnp.float32),
            pltpu.SemaphoreType.DMA,
            pltpu.SemaphoreType.DMA,
            pltpu.SemaphoreType.DMA,
            pltpu.SemaphoreType.DMA,
        ),
        compiler_params=_SC_PARAMS,
    )
    if first:
        return fn(pm_row, cm_row, tr, src_init, v)
    return fn(pm_row, cm_row, tr, src_prev, src_init, v)


SRCP = 112896
TPW = 98
PW = TPW * 128
TSUB = 7
SUBP = TSUB * 128
NSUB = TPW // TSUB
NTILE = T * N // 128


def _fin_body(l_hbm, sall_hbm, o_hbm, bufa, bufb, idsv, rows, tbuf, sem, sem2):
    w = _wid()
    tile0 = jnp.minimum(w * TPW, NTILE - TPW)
    p_lo = tile0 * 128
    t_lo = ((p_lo >= N).astype(jnp.int32) + (p_lo >= 2 * N).astype(jnp.int32)
            + (p_lo >= 3 * N).astype(jnp.int32))
    i_lo = p_lo - t_lo * N
    tbound = (t_lo + 1) * N
    shift = tbound - p_lo
    tb = jnp.minimum(t_lo + 1, 3)
    da = pltpu.async_copy(sall_hbm.at[pl.ds(t_lo * SRCP + i_lo, PW)], bufa, sem)
    db = pltpu.async_copy(sall_hbm.at[pl.ds(tb * SRCP, PW)], bufb, sem2)
    da.wait()
    db.wait()
    for g in range(PW // 16):
        k16 = g * 16 + _iota16()
        p16 = p_lo + k16
        a16 = bufa[pl.ds(g * 16, 16)]
        b16 = plsc.load_gather(bufb, [jnp.maximum(k16 - shift, 0)])
        idsv[pl.ds(g * 16, 16)] = jnp.where(p16 < tbound, a16, b16)
    for sub in range(NSUB):
        pltpu.async_copy(
            l_hbm.at[idsv.at[pl.ds(sub * SUBP, SUBP)]], rows, sem
        ).wait()

        def col_blk(blk, carry):
            def col_c(c, carry2):
                cr = c // 8
                cc = c - cr * 8
                cvec = c + 0 * _iota16()
                for lv in range(8):
                    lane16 = lv * 16 + _iota16()
                    v = plsc.load_gather(rows, [blk * 128 + lane16, cvec])
                    tbuf[cr, pl.ds(blk * 1024 + cc * 128 + lv * 16, 16)] = v
                return carry2

            return lax.fori_loop(0, NCLS, col_c, carry)

        lax.fori_loop(0, TSUB, col_blk, 0)
        pltpu.sync_copy(
            tbuf, o_hbm.at[:, pl.ds((tile0 + sub * TSUB) * 1024, TSUB * 1024)]
        )


def _final_gather(logits_tab, src_all):
    fn = _mpmd._mpmd_map(
        ((_MESH, _fin_body),),
        (jax.ShapeDtypeStruct((3, NTILE * 1024), jnp.float32),),
        scratch_types=(
            pltpu.VMEM((PW,), jnp.int32),
            pltpu.VMEM((PW,), jnp.int32),
            pltpu.VMEM((PW,), jnp.int32),
            pltpu.VMEM((SUBP, LW), jnp.float32),
            pltpu.VMEM((3, TSUB * 1024), jnp.float32),
            pltpu.SemaphoreType.DMA,
            pltpu.SemaphoreType.DMA,
        ),
        compiler_params=_SC_LINEAR,
    )
    return fn(logits_tab, src_all)[0]


def kernel(feats, cur_match, prev_match, W_fem1, W_fem2, W_rrm1, W_rrm2, W_cls):
    f32 = jnp.float32
    cm = cur_match.astype(jnp.int32)
    pm = prev_match.astype(jnp.int32)

    pm_p = jnp.pad(pm, ((0, 0), (0, MP2 - M)))
    cm_p = jnp.pad(cm, ((0, 0), (0, MP - M)), constant_values=N)
    lbi_flat = jnp.pad(cm[:, THR:], ((0, 0), (0, RP - R))).reshape(-1)
    feats0p = jnp.pad(feats[0], ((0, BMF * (NV // BMF) - N), (0, 8 - DIN)))
    w1p_fem = jnp.pad(W_fem1, ((0, 8 - DIN), (0, 0)))
    w1e_rrm = jnp.kron(jnp.eye(32, dtype=f32), W_rrm1)
    w2p_fem = jnp.pad(W_fem2, ((0, 0), (0, 128 - DOUT)))
    w2p_rrm = jnp.pad(W_rrm2, ((0, 0), (0, 128 - DOUT)))
    wce = jnp.kron(
        jnp.eye(4, dtype=f32),
        jnp.pad(W_cls, ((0, 128 - DOUT), (0, LW - NCLS))),
    )
    src_init = SENT + (jnp.arange(SRCL, dtype=jnp.int32) & 255)

    v, lp0 = _fem(feats0p, w1p_fem, w2p_fem, wce)

    ffl1 = jnp.reshape(feats[1], (N * DIN,))
    ffl2 = jnp.reshape(feats[2], (N * DIN,))
    ffl3 = jnp.reshape(feats[3], (N * DIN,))
    gf_flat = _gather_feats(ffl1, ffl2, ffl3, lbi_flat)
    gf_r = jnp.reshape(gf_flat, (3 * RP * DIN // 128, 128))
    tr = _mlp(gf_r, w1e_rrm, w2p_rrm, 3 * RP, None, 120, 8)

    src1, v = _compose_fuse(1, pm_p[0], cm_p[0], tr, None, src_init, v)
    src2, v = _compose_fuse(2, pm_p[1], cm_p[1], tr, src1, src_init, v)
    src3, v = _compose_fuse(3, pm_p[2], cm_p[2], tr, src2, src_init, v)

    logits_packed = _cls(v, wce, lp0)
    logits_tab = jnp.reshape(logits_packed, (NV, LW))

    src_all = jnp.concatenate([
        jnp.arange(SRCP, dtype=jnp.int32),
        jnp.pad(src1, (0, SRCP - SRCL)),
        jnp.pad(src2, (0, SRCP - SRCL)),
        jnp.pad(src3, (0, SRCP - SRCL)),
    ])
    obuf = _final_gather(logits_tab, src_all).reshape(3, NTILE, 8, 128)
    out = jnp.transpose(obuf, (1, 3, 0, 2)).reshape(T * N, 24)
    return out[:, :NCLS]
